# baseline (speedup 1.0000x reference)
import functools

import jax
import jax.numpy as jnp
from jax import lax
from jax.experimental import pallas as pl
from jax.experimental.pallas import tpu as pltpu
from jax.experimental.pallas import tpu_sc as plsc

B, S, HID = 1, 2048, 1024
NE, NG = 8, 2
FF_E, FF_S = 512, 2048
ROUTE_SCALE = 2.5

N = B * S
TOPK = 2
NSLOT = N * TOPK
BT = 256
BLK = 512
NB = NSLOT // BLK + NE - 1
NS_PAD = NB * BLK

NW = 32
SLOTS_W = NSLOT // NW
TOK_W = N // NW
DC = 32
NDC = SLOTS_W // DC
_CT = 16
NCC = TOK_W // _CT


def _router_body(x_ref, g_ref, b_ref, wsu_ref, wsd_ref,
                 sh_ref, dest_ref, wrep_ref, be_ref, nact_ref,
                 cnt_ref, tki_s, rank_s):
    i = pl.program_id(0)

    @pl.when(i == 0)
    def _init():
        cnt_ref[...] = jnp.zeros((1, NE), jnp.float32)

    @pl.when(i < N // BT)
    def _block():
        _router_block(x_ref, g_ref, b_ref, wsu_ref, wsd_ref,
                      sh_ref, wrep_ref, cnt_ref, tki_s, rank_s, i)

    @pl.when(i == N // BT)
    def _finalize():
        tot = cnt_ref[...]
        pad = jnp.floor((tot + (BLK - 1)) / BLK) * BLK
        e_i = lax.broadcasted_iota(jnp.int32, (NE, NE), 0)
        e_j = lax.broadcasted_iota(jnp.int32, (NE, NE), 1)
        tri8 = jnp.where(e_i < e_j, 1.0, 0.0)
        off = lax.dot_general(pad, tri8, (((1,), (0,)), ((), ())),
                              preferred_element_type=jnp.float32)
        incl_blk = (off + pad) / BLK
        b_iota = lax.broadcasted_iota(jnp.int32, (1, NB), 1).astype(
            jnp.float32)
        col8f = lax.broadcasted_iota(jnp.int32, (1, NE), 1)
        be = jnp.zeros((1, NB), jnp.float32)
        for e in range(NE):
            incl_e = jnp.sum(jnp.where(col8f == e, incl_blk, 0.0))
            be = be + jnp.where(b_iota >= incl_e, 1.0, 0.0)
        be_ref[...] = jnp.minimum(be, NE - 1).astype(jnp.int32)
        nact_ref[...] = (jnp.sum(pad, axis=1, keepdims=True) / BLK
                         ).astype(jnp.int32)

        col8b = lax.broadcasted_iota(jnp.int32, (N, NE), 1)
        c1a = tki_s[:, 0:1]
        c2a = tki_s[:, 1:2]
        o1 = jnp.sum(jnp.where(col8b == c1a, off, 0.0), axis=1,
                     keepdims=True)
        o2 = jnp.sum(jnp.where(col8b == c2a, off, 0.0), axis=1,
                     keepdims=True)
        dest_ref[...] = (jnp.concatenate([o1, o2], axis=1).astype(jnp.int32)
                         + rank_s[...])


def _router_block(x_ref, g_ref, b_ref, wsu_ref, wsd_ref,
                  sh_ref, wrep_ref, cnt_ref, tki_s, rank_s, i):
    x = x_ref[...]
    logits = lax.dot_general(x, g_ref[...], (((1,), (1,)), ((), ())),
                             preferred_element_type=jnp.float32)
    scores = jax.nn.sigmoid(logits)
    sb = scores + b_ref[...]

    col4 = lax.broadcasted_iota(jnp.int32, (BT, NE // NG), 1)

    def top2sum(v):
        m1 = jnp.max(v, axis=1, keepdims=True)
        first = jnp.min(jnp.where(v == m1, col4, NE), axis=1, keepdims=True)
        v2 = jnp.where(col4 == first, -1e30, v)
        m2 = jnp.max(v2, axis=1, keepdims=True)
        return m1 + m2

    gs0 = top2sum(sb[:, : NE // NG])
    gs1 = top2sum(sb[:, NE // NG:])
    g0_wins = gs0 >= gs1

    col8 = lax.broadcasted_iota(jnp.int32, (BT, NE), 1)
    in_g0 = jnp.where(col8 < (NE // NG), 1.0, 0.0)
    g0w = jnp.where(g0_wins, 1.0, 0.0)
    mask = g0w * in_g0 + (1.0 - g0w) * (1.0 - in_g0)
    ms = jnp.where(mask > 0.5, sb, -1e9)

    m1 = jnp.max(ms, axis=1, keepdims=True)
    c1 = jnp.min(jnp.where(ms == m1, col8, NE), axis=1, keepdims=True)
    ms2 = jnp.where(col8 == c1, -1e30, ms)
    m2 = jnp.max(ms2, axis=1, keepdims=True)
    c2 = jnp.min(jnp.where(ms2 == m2, col8, NE), axis=1, keepdims=True)

    w1s = jnp.sum(jnp.where(col8 == c1, scores, 0.0), axis=1, keepdims=True)
    w2s = jnp.sum(jnp.where(col8 == c2, scores, 0.0), axis=1, keepdims=True)
    denom = w1s + w2s + 1e-20
    ones128 = jnp.full((1, 128), 1.0, jnp.float32)
    wrep_ref[...] = jnp.concatenate(
        [(w1s / denom * ROUTE_SCALE) * ones128,
         (w2s / denom * ROUTE_SCALE) * ones128], axis=1).reshape(2 * BT, 128)
    tki_s[pl.ds(i * BT, BT), :] = jnp.concatenate([c1, c2], axis=1)

    oh = (jnp.where(col8 == c1, 1.0, 0.0) + jnp.where(col8 == c2, 1.0, 0.0))
    ri = lax.broadcasted_iota(jnp.int32, (BT, BT), 0)
    ci = lax.broadcasted_iota(jnp.int32, (BT, BT), 1)
    tri = jnp.where(ri > ci, 1.0, 0.0)
    excl = lax.dot_general(tri, oh, (((1,), (0,)), ((), ())),
                           preferred_element_type=jnp.float32)
    glob = excl + cnt_ref[...]
    r1 = jnp.sum(jnp.where(col8 == c1, glob, 0.0), axis=1, keepdims=True)
    r2 = jnp.sum(jnp.where(col8 == c2, glob, 0.0), axis=1, keepdims=True)
    rank_s[pl.ds(i * BT, BT), :] = jnp.concatenate(
        [r1, r2], axis=1).astype(jnp.int32)
    cnt_ref[...] = cnt_ref[...] + jnp.sum(oh, axis=0, keepdims=True)

    xb = x.astype(jnp.bfloat16)
    h = lax.dot_general(xb, wsu_ref[...].astype(jnp.bfloat16),
                        (((1,), (1,)), ((), ())),
                        preferred_element_type=jnp.float32)
    hrelu = jnp.maximum(h, 0.0).astype(jnp.bfloat16)
    sh_ref[...] = lax.dot_general(hrelu,
                                  wsd_ref[...].astype(jnp.bfloat16),
                                  (((1,), (1,)), ((), ())),
                                  preferred_element_type=jnp.float32)


@jax.jit
def _stage1(flat, gate_weight, bias2d, wsu, wsd):
    nblk = N // BT
    last = nblk - 1
    return pl.pallas_call(
        _router_body,
        grid=(nblk + 1,),
        in_specs=[
            pl.BlockSpec((BT, HID), lambda i: (jnp.minimum(i, last), 0)),
            pl.BlockSpec((NE, HID), lambda i: (0, 0)),
            pl.BlockSpec((1, NE), lambda i: (0, 0)),
            pl.BlockSpec((FF_S, HID), lambda i: (0, 0)),
            pl.BlockSpec((HID, FF_S), lambda i: (0, 0)),
        ],
        out_specs=[
            pl.BlockSpec((BT, HID), lambda i: (jnp.minimum(i, last), 0)),
            pl.BlockSpec((N, TOPK), lambda i: (0, 0)),
            pl.BlockSpec((2 * BT, 128), lambda i: (jnp.minimum(i, last), 0)),
            pl.BlockSpec((1, NB), lambda i: (0, 0)),
            pl.BlockSpec((1, 1), lambda i: (0, 0)),
        ],
        out_shape=[
            jax.ShapeDtypeStruct((N, HID), jnp.float32),
            jax.ShapeDtypeStruct((N, TOPK), jnp.int32),
            jax.ShapeDtypeStruct((NSLOT, 128), jnp.float32),
            jax.ShapeDtypeStruct((1, NB), jnp.int32),
            jax.ShapeDtypeStruct((1, 1), jnp.int32),
        ],
        scratch_shapes=[
            pltpu.VMEM((1, NE), jnp.float32),
            pltpu.VMEM((N, TOPK), jnp.int32),
            pltpu.VMEM((N, TOPK), jnp.int32),
        ],
    )(flat, gate_weight, bias2d, wsu, wsd)


@functools.cache
def _get_dispatch():
    mesh = plsc.VectorSubcoreMesh(core_axis_name="c", subcore_axis_name="s")
    return functools.partial(
        pl.kernel,
        mesh=mesh,
        out_type=[
            jax.ShapeDtypeStruct((NS_PAD, HID), jnp.float32),
            jax.ShapeDtypeStruct((NS_PAD, 128), jnp.float32),
        ],
        scratch_types=[
            pltpu.VMEM((NDC, DC), jnp.int32),
            pltpu.VMEM((NDC, DC), jnp.int32),
            pltpu.VMEM((SLOTS_W, 128), jnp.float32),
            pltpu.VMEM((DC, HID), jnp.float32),
            pltpu.VMEM((DC, HID), jnp.float32),
            pltpu.SemaphoreType.DMA,
            pltpu.SemaphoreType.DMA,
            pltpu.SemaphoreType.DMA,
        ],
    )(_dispatch_body)


def _dispatch_body(flat, tok2, dest2, wrep, xs, ws,
                   tok2d, dest2d, w_v, rows0, rows1, gsem, ssem, csem):
    wid = lax.axis_index("s") * 2 + lax.axis_index("c")
    rbase = wid * NDC
    pltpu.async_copy(tok2.at[pl.ds(rbase, NDC)], tok2d, csem).wait()
    pltpu.async_copy(dest2.at[pl.ds(rbase, NDC)], dest2d, csem).wait()
    wv_copy = pltpu.async_copy(wrep.at[pl.ds(wid * SLOTS_W, SLOTS_W)], w_v,
                               csem)

    bufs = [rows0, rows1]
    g = pltpu.async_copy(flat.at[tok2d.at[0]], bufs[0], gsem)
    wv_copy.wait()
    s_prev = None
    wscat = []
    for c in range(NDC):
        g.wait()
        if s_prev is not None:
            s_prev.wait()
        if c + 1 < NDC:
            g = pltpu.async_copy(flat.at[tok2d.at[c + 1]],
                                 bufs[(c + 1) % 2], gsem)
        s_prev = pltpu.async_copy(bufs[c % 2], xs.at[dest2d.at[c]], ssem)
        wscat.append(
            pltpu.async_copy(w_v.at[pl.ds(c * DC, DC)], ws.at[dest2d.at[c]],
                             csem))
    s_prev.wait()
    for h in wscat:
        h.wait()


def _gemm_body(be_ref, nact_ref, x_ref, w1_ref, w2_ref, ws_ref, ye_ref):
    i = pl.program_id(0)

    @pl.when(i < nact_ref[0])
    def _compute():
        xb = x_ref[...].astype(jnp.bfloat16)
        h = lax.dot_general(xb, w1_ref[0].astype(jnp.bfloat16),
                            (((1,), (1,)), ((), ())),
                            preferred_element_type=jnp.float32)
        hr = jnp.maximum(h, 0.0)
        hsq = (hr * hr).astype(jnp.bfloat16)
        ye = lax.dot_general(hsq, w2_ref[0].astype(jnp.bfloat16),
                             (((1,), (1,)), ((), ())),
                             preferred_element_type=jnp.float32)
        ye_ref[...] = ye * ws_ref[:, 0:1]


@jax.jit
def _stage3(be, nact, xs, w1b, w2b, ws):
    def _act(i, na):
        return jnp.minimum(i, na[0] - 1)

    grid_spec = pltpu.PrefetchScalarGridSpec(
        num_scalar_prefetch=2,
        grid=(NB,),
        in_specs=[
            pl.BlockSpec((BLK, HID), lambda i, be, na: (_act(i, na), 0)),
            pl.BlockSpec((1, FF_E, HID),
                         lambda i, be, na: (be[_act(i, na)], 0, 0)),
            pl.BlockSpec((1, HID, FF_E),
                         lambda i, be, na: (be[_act(i, na)], 0, 0)),
            pl.BlockSpec((BLK, 128), lambda i, be, na: (_act(i, na), 0)),
        ],
        out_specs=pl.BlockSpec((BLK, HID),
                               lambda i, be, na: (_act(i, na), 0)),
    )
    return pl.pallas_call(
        _gemm_body,
        grid_spec=grid_spec,
        out_shape=jax.ShapeDtypeStruct((NS_PAD, HID), jnp.float32),
    )(be, nact, xs, w1b, w2b, ws)


@functools.cache
def _get_combine():
    mesh = plsc.VectorSubcoreMesh(core_axis_name="c", subcore_axis_name="s")
    return functools.partial(
        pl.kernel,
        mesh=mesh,
        out_type=jax.ShapeDtypeStruct((N, HID), jnp.float32),
        scratch_types=[
            pltpu.VMEM((NCC, _CT), jnp.int32),
            pltpu.VMEM((NCC, _CT), jnp.int32),
            pltpu.VMEM((_CT, HID), jnp.float32),
            pltpu.VMEM((_CT, HID), jnp.float32),
            pltpu.VMEM((_CT, HID), jnp.float32),
            pltpu.VMEM((_CT, HID), jnp.float32),
            pltpu.VMEM((_CT, HID), jnp.float32),
            pltpu.VMEM((_CT, HID), jnp.float32),
            pltpu.SemaphoreType.DMA,
            pltpu.SemaphoreType.DMA,
            pltpu.SemaphoreType.DMA,
        ],
    )(_combine_body)


def _combine_body(ye, de2, do2, shared, out,
                  de2d, do2d, be0, be1, bo0, bo1, sh_v, out_v,
                  gsem, osem, csem):
    wid = lax.axis_index("s") * 2 + lax.axis_index("c")
    base_t = wid * TOK_W
    rbase = wid * NCC
    pltpu.async_copy(de2.at[pl.ds(rbase, NCC)], de2d, csem).wait()
    pltpu.async_copy(do2.at[pl.ds(rbase, NCC)], do2d, csem).wait()

    ebufs = [be0, be1]
    obufs = [bo0, bo1]
    ge = pltpu.async_copy(ye.at[de2d.at[0]], ebufs[0], gsem)
    go = pltpu.async_copy(ye.at[do2d.at[0]], obufs[0], gsem)
    o_prev = None
    for c in range(NCC):
        pltpu.async_copy(shared.at[pl.ds(base_t + c * _CT, _CT)],
                         sh_v, csem).wait()
        ge.wait()
        go.wait()
        if c + 1 < NCC:
            ge = pltpu.async_copy(ye.at[de2d.at[c + 1]],
                                  ebufs[(c + 1) % 2], gsem)
            go = pltpu.async_copy(ye.at[do2d.at[c + 1]],
                                  obufs[(c + 1) % 2], gsem)
        if o_prev is not None:
            o_prev.wait()
        ev = ebufs[c % 2]
        ov = obufs[c % 2]

        def body(tt, _):
            for l in range(HID // 16):
                seg = pl.ds(l * 16, 16)
                out_v[tt, seg] = (sh_v[tt, seg] + ev[tt, seg] + ov[tt, seg])
            return 0

        lax.fori_loop(0, _CT, body, 0, unroll=2)
        o_prev = pltpu.async_copy(out_v, out.at[pl.ds(base_t + c * _CT, _CT)],
                                  osem)
    o_prev.wait()


def kernel(hidden_states, gate_weight, e_score_correction_bias,
           w1, w2, ws_up, ws_down):
    Bx, Sx, D = hidden_states.shape
    flat = hidden_states.reshape(-1, D)
    sh, dest, wrep, be, nact = _stage1(
        flat, gate_weight, e_score_correction_bias.reshape(1, NE),
        ws_up, ws_down)
    tok2 = (jnp.arange(NSLOT, dtype=jnp.int32) >> 1).reshape(NW * NDC, DC)
    dest2 = dest.reshape(NW * NDC, DC)
    de2 = dest[:, 0].reshape(NW * NCC, _CT)
    do2 = dest[:, 1].reshape(NW * NCC, _CT)
    xs, ws = _get_dispatch()(flat, tok2, dest2, wrep)
    ye = _stage3(be.reshape(-1), nact.reshape(-1), xs, w1, w2, ws)
    out = _get_combine()(ye, de2, do2, sh)
    return out.reshape(Bx, Sx, D)


# speedup vs baseline: 1.0057x; 1.0057x over previous
import functools

import jax
import jax.numpy as jnp
from jax import lax
from jax.experimental import pallas as pl
from jax.experimental.pallas import tpu as pltpu
from jax.experimental.pallas import tpu_sc as plsc

B, S, HID = 1, 2048, 1024
NE, NG = 8, 2
FF_E, FF_S = 512, 2048
ROUTE_SCALE = 2.5

N = B * S
TOPK = 2
NSLOT = N * TOPK
BT = 256
BLK = 512
NB = NSLOT // BLK + NE - 1
NS_PAD = NB * BLK

NW = 32
SLOTS_W = NSLOT // NW
TOK_W = N // NW
DC = 32
NDC = SLOTS_W // DC
_CT = 16
NCC = TOK_W // _CT


def _router_body(x_ref, g_ref, b_ref, wsu_ref, wsd_ref,
                 sh_ref, dest_ref, wrep_ref, be_ref, nact_ref,
                 cnt_ref, tki_s, rank_s):
    i = pl.program_id(0)

    @pl.when(i == 0)
    def _init():
        cnt_ref[...] = jnp.zeros((1, NE), jnp.float32)

    @pl.when(i < N // BT)
    def _block():
        _router_block(x_ref, g_ref, b_ref, wsu_ref, wsd_ref,
                      sh_ref, wrep_ref, cnt_ref, tki_s, rank_s, i)

    @pl.when(i == N // BT)
    def _finalize():
        tot = cnt_ref[...]
        pad = jnp.floor((tot + (BLK - 1)) / BLK) * BLK
        e_i = lax.broadcasted_iota(jnp.int32, (NE, NE), 0)
        e_j = lax.broadcasted_iota(jnp.int32, (NE, NE), 1)
        tri8 = jnp.where(e_i < e_j, 1.0, 0.0)
        off = lax.dot_general(pad, tri8, (((1,), (0,)), ((), ())),
                              preferred_element_type=jnp.float32)
        incl_blk = (off + pad) / BLK
        b_iota = lax.broadcasted_iota(jnp.int32, (1, NB), 1).astype(
            jnp.float32)
        col8f = lax.broadcasted_iota(jnp.int32, (1, NE), 1)
        be = jnp.zeros((1, NB), jnp.float32)
        for e in range(NE):
            incl_e = jnp.sum(jnp.where(col8f == e, incl_blk, 0.0))
            be = be + jnp.where(b_iota >= incl_e, 1.0, 0.0)
        be_ref[...] = jnp.minimum(be, NE - 1).astype(jnp.int32)
        nact_ref[...] = (jnp.sum(pad, axis=1, keepdims=True) / BLK
                         ).astype(jnp.int32)

        col8b = lax.broadcasted_iota(jnp.int32, (N, NE), 1)
        c1a = tki_s[:, 0:1]
        c2a = tki_s[:, 1:2]
        o1 = jnp.sum(jnp.where(col8b == c1a, off, 0.0), axis=1,
                     keepdims=True)
        o2 = jnp.sum(jnp.where(col8b == c2a, off, 0.0), axis=1,
                     keepdims=True)
        dest_ref[...] = (jnp.concatenate([o1, o2], axis=1).astype(jnp.int32)
                         + rank_s[...])


def _router_block(x_ref, g_ref, b_ref, wsu_ref, wsd_ref,
                  sh_ref, wrep_ref, cnt_ref, tki_s, rank_s, i):
    x = x_ref[...]
    logits = lax.dot_general(x, g_ref[...], (((1,), (1,)), ((), ())),
                             preferred_element_type=jnp.float32)
    scores = jax.nn.sigmoid(logits)
    sb = scores + b_ref[...]

    col4 = lax.broadcasted_iota(jnp.int32, (BT, NE // NG), 1)

    def top2sum(v):
        m1 = jnp.max(v, axis=1, keepdims=True)
        first = jnp.min(jnp.where(v == m1, col4, NE), axis=1, keepdims=True)
        v2 = jnp.where(col4 == first, -1e30, v)
        m2 = jnp.max(v2, axis=1, keepdims=True)
        return m1 + m2

    gs0 = top2sum(sb[:, : NE // NG])
    gs1 = top2sum(sb[:, NE // NG:])
    g0_wins = gs0 >= gs1

    col8 = lax.broadcasted_iota(jnp.int32, (BT, NE), 1)
    in_g0 = jnp.where(col8 < (NE // NG), 1.0, 0.0)
    g0w = jnp.where(g0_wins, 1.0, 0.0)
    mask = g0w * in_g0 + (1.0 - g0w) * (1.0 - in_g0)
    ms = jnp.where(mask > 0.5, sb, -1e9)

    m1 = jnp.max(ms, axis=1, keepdims=True)
    c1 = jnp.min(jnp.where(ms == m1, col8, NE), axis=1, keepdims=True)
    ms2 = jnp.where(col8 == c1, -1e30, ms)
    m2 = jnp.max(ms2, axis=1, keepdims=True)
    c2 = jnp.min(jnp.where(ms2 == m2, col8, NE), axis=1, keepdims=True)

    w1s = jnp.sum(jnp.where(col8 == c1, scores, 0.0), axis=1, keepdims=True)
    w2s = jnp.sum(jnp.where(col8 == c2, scores, 0.0), axis=1, keepdims=True)
    denom = w1s + w2s + 1e-20
    ones128 = jnp.full((1, 128), 1.0, jnp.float32)
    wrep_ref[...] = jnp.concatenate(
        [(w1s / denom * ROUTE_SCALE) * ones128,
         (w2s / denom * ROUTE_SCALE) * ones128], axis=1).reshape(2 * BT, 128)
    tki_s[pl.ds(i * BT, BT), :] = jnp.concatenate([c1, c2], axis=1)

    oh = (jnp.where(col8 == c1, 1.0, 0.0) + jnp.where(col8 == c2, 1.0, 0.0))
    ri = lax.broadcasted_iota(jnp.int32, (BT, BT), 0)
    ci = lax.broadcasted_iota(jnp.int32, (BT, BT), 1)
    tri = jnp.where(ri > ci, 1.0, 0.0)
    excl = lax.dot_general(tri, oh, (((1,), (0,)), ((), ())),
                           preferred_element_type=jnp.float32)
    glob = excl + cnt_ref[...]
    r1 = jnp.sum(jnp.where(col8 == c1, glob, 0.0), axis=1, keepdims=True)
    r2 = jnp.sum(jnp.where(col8 == c2, glob, 0.0), axis=1, keepdims=True)
    rank_s[pl.ds(i * BT, BT), :] = jnp.concatenate(
        [r1, r2], axis=1).astype(jnp.int32)
    cnt_ref[...] = cnt_ref[...] + jnp.sum(oh, axis=0, keepdims=True)

    h = lax.dot_general(x, wsu_ref[...], (((1,), (1,)), ((), ())),
                        preferred_element_type=jnp.float32)
    hrelu = jnp.maximum(h, 0.0)
    sh_ref[...] = lax.dot_general(hrelu, wsd_ref[...],
                                  (((1,), (1,)), ((), ())),
                                  preferred_element_type=jnp.float32)


@jax.jit
def _stage1(flat, gate_weight, bias2d, wsu, wsd):
    nblk = N // BT
    last = nblk - 1
    return pl.pallas_call(
        _router_body,
        grid=(nblk + 1,),
        in_specs=[
            pl.BlockSpec((BT, HID), lambda i: (jnp.minimum(i, last), 0)),
            pl.BlockSpec((NE, HID), lambda i: (0, 0)),
            pl.BlockSpec((1, NE), lambda i: (0, 0)),
            pl.BlockSpec((FF_S, HID), lambda i: (0, 0)),
            pl.BlockSpec((HID, FF_S), lambda i: (0, 0)),
        ],
        out_specs=[
            pl.BlockSpec((BT, HID), lambda i: (jnp.minimum(i, last), 0)),
            pl.BlockSpec((N, TOPK), lambda i: (0, 0)),
            pl.BlockSpec((2 * BT, 128), lambda i: (jnp.minimum(i, last), 0)),
            pl.BlockSpec((1, NB), lambda i: (0, 0)),
            pl.BlockSpec((1, 1), lambda i: (0, 0)),
        ],
        out_shape=[
            jax.ShapeDtypeStruct((N, HID), jnp.float32),
            jax.ShapeDtypeStruct((N, TOPK), jnp.int32),
            jax.ShapeDtypeStruct((NSLOT, 128), jnp.float32),
            jax.ShapeDtypeStruct((1, NB), jnp.int32),
            jax.ShapeDtypeStruct((1, 1), jnp.int32),
        ],
        scratch_shapes=[
            pltpu.VMEM((1, NE), jnp.float32),
            pltpu.VMEM((N, TOPK), jnp.int32),
            pltpu.VMEM((N, TOPK), jnp.int32),
        ],
    )(flat, gate_weight, bias2d, wsu, wsd)


@functools.cache
def _get_dispatch():
    mesh = plsc.VectorSubcoreMesh(core_axis_name="c", subcore_axis_name="s")
    return functools.partial(
        pl.kernel,
        mesh=mesh,
        out_type=[
            jax.ShapeDtypeStruct((NS_PAD, HID), jnp.float32),
            jax.ShapeDtypeStruct((NS_PAD, 128), jnp.float32),
        ],
        scratch_types=[
            pltpu.VMEM((NDC, DC), jnp.int32),
            pltpu.VMEM((NDC, DC), jnp.int32),
            pltpu.VMEM((SLOTS_W, 128), jnp.float32),
            pltpu.VMEM((DC, HID), jnp.float32),
            pltpu.VMEM((DC, HID), jnp.float32),
            pltpu.SemaphoreType.DMA,
            pltpu.SemaphoreType.DMA,
            pltpu.SemaphoreType.DMA,
        ],
    )(_dispatch_body)


def _dispatch_body(flat, tok2, dest2, wrep, xs, ws,
                   tok2d, dest2d, w_v, rows0, rows1, gsem, ssem, csem):
    wid = lax.axis_index("s") * 2 + lax.axis_index("c")
    rbase = wid * NDC
    pltpu.async_copy(tok2.at[pl.ds(rbase, NDC)], tok2d, csem).wait()
    pltpu.async_copy(dest2.at[pl.ds(rbase, NDC)], dest2d, csem).wait()
    wv_copy = pltpu.async_copy(wrep.at[pl.ds(wid * SLOTS_W, SLOTS_W)], w_v,
                               csem)

    bufs = [rows0, rows1]
    g = pltpu.async_copy(flat.at[tok2d.at[0]], bufs[0], gsem)
    wv_copy.wait()
    s_prev = None
    wscat = []
    for c in range(NDC):
        g.wait()
        if s_prev is not None:
            s_prev.wait()
        if c + 1 < NDC:
            g = pltpu.async_copy(flat.at[tok2d.at[c + 1]],
                                 bufs[(c + 1) % 2], gsem)
        s_prev = pltpu.async_copy(bufs[c % 2], xs.at[dest2d.at[c]], ssem)
        wscat.append(
            pltpu.async_copy(w_v.at[pl.ds(c * DC, DC)], ws.at[dest2d.at[c]],
                             csem))
    s_prev.wait()
    for h in wscat:
        h.wait()


def _gemm_body(be_ref, nact_ref, x_ref, w1_ref, w2_ref, ws_ref, ye_ref):
    i = pl.program_id(0)

    @pl.when(i < nact_ref[0])
    def _compute():
        x = x_ref[...]
        h = lax.dot_general(x, w1_ref[0], (((1,), (1,)), ((), ())),
                            preferred_element_type=jnp.float32)
        hr = jnp.maximum(h, 0.0)
        hsq = hr * hr
        ye = lax.dot_general(hsq, w2_ref[0], (((1,), (1,)), ((), ())),
                             preferred_element_type=jnp.float32)
        ye_ref[...] = ye * ws_ref[:, 0:1]


@jax.jit
def _stage3(be, nact, xs, w1b, w2b, ws):
    def _act(i, na):
        return jnp.minimum(i, na[0] - 1)

    grid_spec = pltpu.PrefetchScalarGridSpec(
        num_scalar_prefetch=2,
        grid=(NB,),
        in_specs=[
            pl.BlockSpec((BLK, HID), lambda i, be, na: (_act(i, na), 0)),
            pl.BlockSpec((1, FF_E, HID),
                         lambda i, be, na: (be[_act(i, na)], 0, 0)),
            pl.BlockSpec((1, HID, FF_E),
                         lambda i, be, na: (be[_act(i, na)], 0, 0)),
            pl.BlockSpec((BLK, 128), lambda i, be, na: (_act(i, na), 0)),
        ],
        out_specs=pl.BlockSpec((BLK, HID),
                               lambda i, be, na: (_act(i, na), 0)),
    )
    return pl.pallas_call(
        _gemm_body,
        grid_spec=grid_spec,
        out_shape=jax.ShapeDtypeStruct((NS_PAD, HID), jnp.float32),
    )(be, nact, xs, w1b, w2b, ws)


@functools.cache
def _get_combine():
    mesh = plsc.VectorSubcoreMesh(core_axis_name="c", subcore_axis_name="s")
    return functools.partial(
        pl.kernel,
        mesh=mesh,
        out_type=jax.ShapeDtypeStruct((N, HID), jnp.float32),
        scratch_types=[
            pltpu.VMEM((NCC, _CT), jnp.int32),
            pltpu.VMEM((NCC, _CT), jnp.int32),
            pltpu.VMEM((_CT, HID), jnp.float32),
            pltpu.VMEM((_CT, HID), jnp.float32),
            pltpu.VMEM((_CT, HID), jnp.float32),
            pltpu.VMEM((_CT, HID), jnp.float32),
            pltpu.VMEM((_CT, HID), jnp.float32),
            pltpu.VMEM((_CT, HID), jnp.float32),
            pltpu.SemaphoreType.DMA,
            pltpu.SemaphoreType.DMA,
            pltpu.SemaphoreType.DMA,
        ],
    )(_combine_body)


def _combine_body(ye, de2, do2, shared, out,
                  de2d, do2d, be0, be1, bo0, bo1, sh_v, out_v,
                  gsem, osem, csem):
    wid = lax.axis_index("s") * 2 + lax.axis_index("c")
    base_t = wid * TOK_W
    rbase = wid * NCC
    pltpu.async_copy(de2.at[pl.ds(rbase, NCC)], de2d, csem).wait()
    pltpu.async_copy(do2.at[pl.ds(rbase, NCC)], do2d, csem).wait()

    ebufs = [be0, be1]
    obufs = [bo0, bo1]
    ge = pltpu.async_copy(ye.at[de2d.at[0]], ebufs[0], gsem)
    go = pltpu.async_copy(ye.at[do2d.at[0]], obufs[0], gsem)
    o_prev = None
    for c in range(NCC):
        pltpu.async_copy(shared.at[pl.ds(base_t + c * _CT, _CT)],
                         sh_v, csem).wait()
        ge.wait()
        go.wait()
        if c + 1 < NCC:
            ge = pltpu.async_copy(ye.at[de2d.at[c + 1]],
                                  ebufs[(c + 1) % 2], gsem)
            go = pltpu.async_copy(ye.at[do2d.at[c + 1]],
                                  obufs[(c + 1) % 2], gsem)
        if o_prev is not None:
            o_prev.wait()
        ev = ebufs[c % 2]
        ov = obufs[c % 2]

        def body(tt, _):
            for l in range(HID // 16):
                seg = pl.ds(l * 16, 16)
                out_v[tt, seg] = (sh_v[tt, seg] + ev[tt, seg] + ov[tt, seg])
            return 0

        lax.fori_loop(0, _CT, body, 0, unroll=2)
        o_prev = pltpu.async_copy(out_v, out.at[pl.ds(base_t + c * _CT, _CT)],
                                  osem)
    o_prev.wait()


def kernel(hidden_states, gate_weight, e_score_correction_bias,
           w1, w2, ws_up, ws_down):
    Bx, Sx, D = hidden_states.shape
    flat = hidden_states.reshape(-1, D)
    sh, dest, wrep, be, nact = _stage1(
        flat, gate_weight, e_score_correction_bias.reshape(1, NE),
        ws_up, ws_down)
    tok2 = (jnp.arange(NSLOT, dtype=jnp.int32) >> 1).reshape(NW * NDC, DC)
    dest2 = dest.reshape(NW * NDC, DC)
    de2 = dest[:, 0].reshape(NW * NCC, _CT)
    do2 = dest[:, 1].reshape(NW * NCC, _CT)
    xs, ws = _get_dispatch()(flat, tok2, dest2, wrep)
    ye = _stage3(be.reshape(-1), nact.reshape(-1), xs, w1, w2, ws)
    out = _get_combine()(ye, de2, do2, sh)
    return out.reshape(Bx, Sx, D)


# speedup vs baseline: 1.0275x; 1.0216x over previous
import functools

import jax
import jax.numpy as jnp
from jax import lax
from jax.experimental import pallas as pl
from jax.experimental.pallas import tpu as pltpu
from jax.experimental.pallas import tpu_sc as plsc

B, S, HID = 1, 2048, 1024
NE, NG = 8, 2
FF_E, FF_S = 512, 2048
ROUTE_SCALE = 2.5

N = B * S
TOPK = 2
NSLOT = N * TOPK
BT = 256
BLK = 512
NB = NSLOT // BLK + NE - 1
NS_PAD = NB * BLK

NW = 32
SLOTS_W = NSLOT // NW
TOK_W = N // NW
DC = 32
NDC = SLOTS_W // DC
_CT = 16
NCC = TOK_W // _CT


def _router_body(x_ref, g_ref, b_ref, wsu_ref, wsd_ref,
                 sh_ref, dest_ref, wrep_ref, be_ref, nact_ref,
                 cnt_ref, tki_s, rank_s):
    i = pl.program_id(0)

    @pl.when(i == 0)
    def _init():
        cnt_ref[...] = jnp.zeros((1, NE), jnp.float32)

    @pl.when(i < N // BT)
    def _block():
        _router_block(x_ref, g_ref, b_ref, wsu_ref, wsd_ref,
                      sh_ref, wrep_ref, cnt_ref, tki_s, rank_s, i)

    @pl.when(i == N // BT)
    def _finalize():
        tot = cnt_ref[...]
        pad = jnp.floor((tot + (BLK - 1)) / BLK) * BLK
        e_i = lax.broadcasted_iota(jnp.int32, (NE, NE), 0)
        e_j = lax.broadcasted_iota(jnp.int32, (NE, NE), 1)
        tri8 = jnp.where(e_i < e_j, 1.0, 0.0)
        off = lax.dot_general(pad, tri8, (((1,), (0,)), ((), ())),
                              preferred_element_type=jnp.float32)
        incl_blk = (off + pad) / BLK
        b_iota = lax.broadcasted_iota(jnp.int32, (1, NB), 1).astype(
            jnp.float32)
        col8f = lax.broadcasted_iota(jnp.int32, (1, NE), 1)
        be = jnp.zeros((1, NB), jnp.float32)
        for e in range(NE):
            incl_e = jnp.sum(jnp.where(col8f == e, incl_blk, 0.0))
            be = be + jnp.where(b_iota >= incl_e, 1.0, 0.0)
        be_ref[...] = jnp.minimum(be, NE - 1).astype(jnp.int32)
        nact_ref[...] = (jnp.sum(pad, axis=1, keepdims=True) / BLK
                         ).astype(jnp.int32)

        col8b = lax.broadcasted_iota(jnp.int32, (N, NE), 1)
        c1a = tki_s[:, 0:1]
        c2a = tki_s[:, 1:2]
        o1 = jnp.sum(jnp.where(col8b == c1a, off, 0.0), axis=1,
                     keepdims=True)
        o2 = jnp.sum(jnp.where(col8b == c2a, off, 0.0), axis=1,
                     keepdims=True)
        dest_ref[...] = (jnp.concatenate([o1, o2], axis=1).astype(jnp.int32)
                         + rank_s[...])


def _router_block(x_ref, g_ref, b_ref, wsu_ref, wsd_ref,
                  sh_ref, wrep_ref, cnt_ref, tki_s, rank_s, i):
    x = x_ref[...]
    logits = lax.dot_general(x, g_ref[...], (((1,), (1,)), ((), ())),
                             preferred_element_type=jnp.float32)
    scores = jax.nn.sigmoid(logits)
    sb = scores + b_ref[...]

    col4 = lax.broadcasted_iota(jnp.int32, (BT, NE // NG), 1)

    def top2sum(v):
        m1 = jnp.max(v, axis=1, keepdims=True)
        first = jnp.min(jnp.where(v == m1, col4, NE), axis=1, keepdims=True)
        v2 = jnp.where(col4 == first, -1e30, v)
        m2 = jnp.max(v2, axis=1, keepdims=True)
        return m1 + m2

    gs0 = top2sum(sb[:, : NE // NG])
    gs1 = top2sum(sb[:, NE // NG:])
    g0_wins = gs0 >= gs1

    col8 = lax.broadcasted_iota(jnp.int32, (BT, NE), 1)
    in_g0 = jnp.where(col8 < (NE // NG), 1.0, 0.0)
    g0w = jnp.where(g0_wins, 1.0, 0.0)
    mask = g0w * in_g0 + (1.0 - g0w) * (1.0 - in_g0)
    ms = jnp.where(mask > 0.5, sb, -1e9)

    m1 = jnp.max(ms, axis=1, keepdims=True)
    c1 = jnp.min(jnp.where(ms == m1, col8, NE), axis=1, keepdims=True)
    ms2 = jnp.where(col8 == c1, -1e30, ms)
    m2 = jnp.max(ms2, axis=1, keepdims=True)
    c2 = jnp.min(jnp.where(ms2 == m2, col8, NE), axis=1, keepdims=True)

    w1s = jnp.sum(jnp.where(col8 == c1, scores, 0.0), axis=1, keepdims=True)
    w2s = jnp.sum(jnp.where(col8 == c2, scores, 0.0), axis=1, keepdims=True)
    denom = w1s + w2s + 1e-20
    ones128 = jnp.full((1, 128), 1.0, jnp.float32)
    wrep_ref[...] = jnp.concatenate(
        [(w1s / denom * ROUTE_SCALE) * ones128,
         (w2s / denom * ROUTE_SCALE) * ones128], axis=1).reshape(2 * BT, 128)
    tki_s[pl.ds(i * BT, BT), :] = jnp.concatenate([c1, c2], axis=1)

    oh = (jnp.where(col8 == c1, 1.0, 0.0) + jnp.where(col8 == c2, 1.0, 0.0))
    ri = lax.broadcasted_iota(jnp.int32, (BT, BT), 0)
    ci = lax.broadcasted_iota(jnp.int32, (BT, BT), 1)
    tri = jnp.where(ri > ci, 1.0, 0.0)
    excl = lax.dot_general(tri, oh, (((1,), (0,)), ((), ())),
                           preferred_element_type=jnp.float32)
    glob = excl + cnt_ref[...]
    r1 = jnp.sum(jnp.where(col8 == c1, glob, 0.0), axis=1, keepdims=True)
    r2 = jnp.sum(jnp.where(col8 == c2, glob, 0.0), axis=1, keepdims=True)
    rank_s[pl.ds(i * BT, BT), :] = jnp.concatenate(
        [r1, r2], axis=1).astype(jnp.int32)
    cnt_ref[...] = cnt_ref[...] + jnp.sum(oh, axis=0, keepdims=True)

    h = lax.dot_general(x, wsu_ref[...], (((1,), (1,)), ((), ())),
                        preferred_element_type=jnp.float32)
    hrelu = jnp.maximum(h, 0.0)
    sh_ref[...] = lax.dot_general(hrelu, wsd_ref[...],
                                  (((1,), (1,)), ((), ())),
                                  preferred_element_type=jnp.float32)


@jax.jit
def _stage1(flat, gate_weight, bias2d, wsu, wsd):
    nblk = N // BT
    last = nblk - 1
    return pl.pallas_call(
        _router_body,
        grid=(nblk + 1,),
        in_specs=[
            pl.BlockSpec((BT, HID), lambda i: (jnp.minimum(i, last), 0)),
            pl.BlockSpec((NE, HID), lambda i: (0, 0)),
            pl.BlockSpec((1, NE), lambda i: (0, 0)),
            pl.BlockSpec((FF_S, HID), lambda i: (0, 0)),
            pl.BlockSpec((HID, FF_S), lambda i: (0, 0)),
        ],
        out_specs=[
            pl.BlockSpec((BT, HID), lambda i: (jnp.minimum(i, last), 0)),
            pl.BlockSpec((N, TOPK), lambda i: (0, 0)),
            pl.BlockSpec((2 * BT, 128), lambda i: (jnp.minimum(i, last), 0)),
            pl.BlockSpec((1, NB), lambda i: (0, 0)),
            pl.BlockSpec((1, 1), lambda i: (0, 0)),
        ],
        out_shape=[
            jax.ShapeDtypeStruct((N, HID), jnp.float32),
            jax.ShapeDtypeStruct((N, TOPK), jnp.int32),
            jax.ShapeDtypeStruct((NSLOT, 128), jnp.float32),
            jax.ShapeDtypeStruct((1, NB), jnp.int32),
            jax.ShapeDtypeStruct((1, 1), jnp.int32),
        ],
        scratch_shapes=[
            pltpu.VMEM((1, NE), jnp.float32),
            pltpu.VMEM((N, TOPK), jnp.int32),
            pltpu.VMEM((N, TOPK), jnp.int32),
        ],
    )(flat, gate_weight, bias2d, wsu, wsd)


@functools.cache
def _get_dispatch():
    mesh = plsc.VectorSubcoreMesh(core_axis_name="c", subcore_axis_name="s")
    return functools.partial(
        pl.kernel,
        mesh=mesh,
        out_type=[
            jax.ShapeDtypeStruct((NS_PAD, HID), jnp.float32),
            jax.ShapeDtypeStruct((NS_PAD, 128), jnp.float32),
        ],
        scratch_types=[
            pltpu.VMEM((NDC, DC), jnp.int32),
            pltpu.VMEM((NDC, DC), jnp.int32),
            pltpu.VMEM((SLOTS_W, 128), jnp.float32),
            pltpu.VMEM((DC, HID), jnp.float32),
            pltpu.VMEM((DC, HID), jnp.float32),
            pltpu.SemaphoreType.DMA,
            pltpu.SemaphoreType.DMA,
            pltpu.SemaphoreType.DMA,
        ],
    )(_dispatch_body)


def _dispatch_body(flat, tok2, dest2, wrep, xs, ws,
                   tok2d, dest2d, w_v, rows0, rows1, gsem, ssem, csem):
    wid = lax.axis_index("s") * 2 + lax.axis_index("c")
    rbase = wid * NDC
    pltpu.async_copy(tok2.at[pl.ds(rbase, NDC)], tok2d, csem).wait()
    pltpu.async_copy(dest2.at[pl.ds(rbase, NDC)], dest2d, csem).wait()
    wv_copy = pltpu.async_copy(wrep.at[pl.ds(wid * SLOTS_W, SLOTS_W)], w_v,
                               csem)

    bufs = [rows0, rows1]
    g = pltpu.async_copy(flat.at[tok2d.at[0]], bufs[0], gsem)
    wv_copy.wait()
    s_prev = None
    wscat = []
    for c in range(NDC):
        g.wait()
        if s_prev is not None:
            s_prev.wait()
        if c + 1 < NDC:
            g = pltpu.async_copy(flat.at[tok2d.at[c + 1]],
                                 bufs[(c + 1) % 2], gsem)
        s_prev = pltpu.async_copy(bufs[c % 2], xs.at[dest2d.at[c]], ssem)
        wscat.append(
            pltpu.async_copy(w_v.at[pl.ds(c * DC, DC)], ws.at[dest2d.at[c]],
                             csem))
    s_prev.wait()
    for h in wscat:
        h.wait()


def _gemm_body(be_ref, nact_ref, x_ref, w1_ref, w2_ref, ws_ref, ye_ref):
    i = pl.program_id(0)

    @pl.when(i < nact_ref[0])
    def _compute():
        x = x_ref[...]
        h = lax.dot_general(x, w1_ref[0], (((1,), (1,)), ((), ())),
                            preferred_element_type=jnp.float32)
        hr = jnp.maximum(h, 0.0)
        hsq = hr * hr
        ye = lax.dot_general(hsq, w2_ref[0], (((1,), (1,)), ((), ())),
                             preferred_element_type=jnp.float32)
        ye_ref[...] = ye * ws_ref[:, 0:1]


@jax.jit
def _stage3(be, nact, xs, w1b, w2b, ws):
    def _act(i, na):
        return jnp.minimum(i, na[0] - 1)

    grid_spec = pltpu.PrefetchScalarGridSpec(
        num_scalar_prefetch=2,
        grid=(NB,),
        in_specs=[
            pl.BlockSpec((BLK, HID), lambda i, be, na: (_act(i, na), 0)),
            pl.BlockSpec((1, FF_E, HID),
                         lambda i, be, na: (be[_act(i, na)], 0, 0)),
            pl.BlockSpec((1, HID, FF_E),
                         lambda i, be, na: (be[_act(i, na)], 0, 0)),
            pl.BlockSpec((BLK, 128), lambda i, be, na: (_act(i, na), 0)),
        ],
        out_specs=pl.BlockSpec((BLK, HID),
                               lambda i, be, na: (_act(i, na), 0)),
    )
    return pl.pallas_call(
        _gemm_body,
        grid_spec=grid_spec,
        out_shape=jax.ShapeDtypeStruct((NS_PAD, HID), jnp.float32),
    )(be, nact, xs, w1b, w2b, ws)


@functools.cache
def _get_combine():
    mesh = plsc.VectorSubcoreMesh(core_axis_name="c", subcore_axis_name="s")
    return functools.partial(
        pl.kernel,
        mesh=mesh,
        out_type=jax.ShapeDtypeStruct((N, HID), jnp.float32),
        scratch_types=[
            pltpu.VMEM((NCC, _CT), jnp.int32),
            pltpu.VMEM((NCC, _CT), jnp.int32),
            pltpu.VMEM((_CT, HID), jnp.float32),
            pltpu.VMEM((_CT, HID), jnp.float32),
            pltpu.VMEM((_CT, HID), jnp.float32),
            pltpu.VMEM((_CT, HID), jnp.float32),
            pltpu.VMEM((_CT, HID), jnp.float32),
            pltpu.VMEM((_CT, HID), jnp.float32),
            pltpu.SemaphoreType.DMA,
            pltpu.SemaphoreType.DMA,
            pltpu.SemaphoreType.DMA,
        ],
    )(_combine_body)


def _combine_body(ye, de2, do2, shared, out,
                  de2d, do2d, be0, be1, bo0, bo1, sh_v, out_v,
                  gsem, osem, csem):
    wid = lax.axis_index("s") * 2 + lax.axis_index("c")
    base_t = wid * TOK_W
    rbase = wid * NCC
    pltpu.async_copy(de2.at[pl.ds(rbase, NCC)], de2d, csem).wait()
    pltpu.async_copy(do2.at[pl.ds(rbase, NCC)], do2d, csem).wait()

    ebufs = [be0, be1]
    obufs = [bo0, bo1]
    ge = pltpu.async_copy(ye.at[de2d.at[0]], ebufs[0], gsem)
    go = pltpu.async_copy(ye.at[do2d.at[0]], obufs[0], gsem)
    o_prev = None
    for c in range(NCC):
        pltpu.async_copy(shared.at[pl.ds(base_t + c * _CT, _CT)],
                         sh_v, csem).wait()
        ge.wait()
        go.wait()
        if c + 1 < NCC:
            ge = pltpu.async_copy(ye.at[de2d.at[c + 1]],
                                  ebufs[(c + 1) % 2], gsem)
            go = pltpu.async_copy(ye.at[do2d.at[c + 1]],
                                  obufs[(c + 1) % 2], gsem)
        if o_prev is not None:
            o_prev.wait()
        ev = ebufs[c % 2]
        ov = obufs[c % 2]

        def body(tt, _):
            for l in range(HID // 16):
                seg = pl.ds(l * 16, 16)
                out_v[tt, seg] = (sh_v[tt, seg] + ev[tt, seg] + ov[tt, seg])
            return 0

        lax.fori_loop(0, _CT, body, 0)
        o_prev = pltpu.async_copy(out_v, out.at[pl.ds(base_t + c * _CT, _CT)],
                                  osem)
    o_prev.wait()


def kernel(hidden_states, gate_weight, e_score_correction_bias,
           w1, w2, ws_up, ws_down):
    Bx, Sx, D = hidden_states.shape
    flat = hidden_states.reshape(-1, D)
    sh, dest, wrep, be, nact = _stage1(
        flat, gate_weight, e_score_correction_bias.reshape(1, NE),
        ws_up, ws_down)
    tok2 = (jnp.arange(NSLOT, dtype=jnp.int32) >> 1).reshape(NW * NDC, DC)
    dest2 = dest.reshape(NW * NDC, DC)
    de2 = dest[:, 0].reshape(NW * NCC, _CT)
    do2 = dest[:, 1].reshape(NW * NCC, _CT)
    xs, ws = _get_dispatch()(flat, tok2, dest2, wrep)
    ye = _stage3(be.reshape(-1), nact.reshape(-1), xs, w1, w2, ws)
    out = _get_combine()(ye, de2, do2, sh)
    return out.reshape(Bx, Sx, D)
